# baseline (speedup 1.0000x reference)
import functools

import jax
import jax.numpy as jnp
from jax import lax
from jax.experimental import pallas as pl
from jax.experimental.pallas import tpu as pltpu
from jax.experimental.pallas import tpu_sc as plsc

N_USERS = 5000
N_NODES = 10000
D = 128
NUM_LAYERS = 3
NE = 320000
B = 4096
REG_W = 1e-4

NC = 2
NS = 16
LANES = 16
NW = NC * NS

ECH = 64
GE = 8
NCH0 = 160
NCH1 = 160
NGRP0 = NCH0 // GE
NGRP1 = NCH1 // GE
NCMAX = max(NCH0, NCH1)
NEDGE_PAD = 16 * (NCH0 + NCH1) * ECH
CH = 128
SUB = 64
ACC_N = 10240
ROWS_PER_TILE = ACC_N // NS

_MESH = plsc.VectorSubcoreMesh(
    core_axis_name="c", subcore_axis_name="s", num_cores=NC, num_subcores=NS)


def _spmm_body(x_hbm, edat_hbm, vals_hbm, part_out,
               acc_sh, edat_a, edat_b, vch_a, vch_b,
               gath_a, gath_b, gath_c, gath_d,
               gsem_a, gsem_b, gsem_c, gsem_d,
               ssem_a, ssem_b, ssem_c, ssem_d, esem):
    c = lax.axis_index("c")
    s = lax.axis_index("s")
    wid = c * NS + s
    ngrp = jnp.where(c == 0, NGRP0, NGRP1)

    def zfill(i, _):
        r = i // (D // LANES)
        col = (i % (D // LANES)) * LANES
        gath_a[r, pl.ds(col, LANES)] = jnp.zeros((LANES,), jnp.float32)
        return 0
    lax.fori_loop(0, ECH * (D // LANES), zfill, 0)
    row0 = s * ROWS_PER_TILE
    for k in range(ROWS_PER_TILE // ECH):
        pltpu.sync_copy(gath_a, acc_sh.at[pl.ds(row0 + k * ECH, ECH)])
    plsc.subcore_barrier()

    pltpu.sync_copy(edat_hbm.at[wid, pl.ds(0, GE)], edat_a)
    pltpu.sync_copy(vals_hbm.at[wid, pl.ds(0, GE)], vch_a)
    pltpu.make_async_copy(x_hbm.at[edat_a.at[0, 1]], gath_a, gsem_a).start()

    ebufs = ((edat_a, vch_a), (edat_b, vch_b))
    gbufs = ((gath_a, gsem_a, ssem_a), (gath_b, gsem_b, ssem_b),
             (gath_c, gsem_c, ssem_c), (gath_d, gsem_d, ssem_d))

    def gpair_body(g2, _):
        for gp in range(2):
            g = g2 * 2 + gp
            eg, vg = ebufs[gp]
            eg_n, vg_n = ebufs[1 - gp]
            have_next_group = g + 1 < ngrp
            for j in range(GE):
                i = g * GE + j
                gath, gsem, ssem = gbufs[j % 4]
                gath_n, gsem_n, ssem_n = gbufs[(j + 1) % 4]

                if j >= 3:
                    pltpu.make_async_copy(
                        gath_n, acc_sh.at[eg.at[j - 3, 0]], ssem_n).wait()
                else:
                    @pl.when(g >= 1)
                    def _():
                        pltpu.make_async_copy(
                            gath_n, acc_sh.at[eg_n.at[GE - 3 + j, 0]],
                            ssem_n).wait()

                if j == 0:
                    @pl.when(have_next_group)
                    def _():
                        gsl = pl.ds((g + 1) * GE, GE)
                        pltpu.make_async_copy(
                            edat_hbm.at[wid, gsl], eg_n, esem).start()
                        pltpu.make_async_copy(
                            vals_hbm.at[wid, gsl], vg_n, esem).start()

                if j < GE - 1:
                    pltpu.make_async_copy(
                        x_hbm.at[eg.at[j + 1, 1]], gath_n, gsem_n).start()
                else:
                    @pl.when(have_next_group)
                    def _():
                        gsl = pl.ds((g + 1) * GE, GE)
                        pltpu.make_async_copy(
                            edat_hbm.at[wid, gsl], eg_n, esem).wait()
                        pltpu.make_async_copy(
                            vals_hbm.at[wid, gsl], vg_n, esem).wait()
                        pltpu.make_async_copy(
                            x_hbm.at[eg_n.at[0, 1]], gath_n, gsem_n).start()

                pltpu.make_async_copy(
                    x_hbm.at[eg.at[j, 1]], gath, gsem).wait()

                def scale(eb, _):
                    val16 = vg[j, 0, pl.ds(eb * LANES, LANES)]
                    for jj in range(LANES):
                        v = val16[jj]
                        e = eb * LANES + jj
                        for d in range(D // LANES):
                            sl = pl.ds(d * LANES, LANES)
                            gath[e, sl] = gath[e, sl] * v
                    return 0
                lax.fori_loop(0, ECH // LANES, scale, 0)

                pltpu.make_async_copy(
                    gath, acc_sh.at[eg.at[j, 0]], ssem).start(add=True)
        return 0
    lax.fori_loop(0, ngrp // 2, gpair_body, 0)

    pltpu.make_async_copy(
        gath_b, acc_sh.at[edat_b.at[GE - 3, 0]], ssem_b).wait()
    pltpu.make_async_copy(
        gath_c, acc_sh.at[edat_b.at[GE - 2, 0]], ssem_c).wait()
    pltpu.make_async_copy(
        gath_d, acc_sh.at[edat_b.at[GE - 1, 0]], ssem_d).wait()

    plsc.subcore_barrier()
    def dump(k, _):
        r = row0 + k * ECH
        pltpu.sync_copy(acc_sh.at[pl.ds(r, ECH)], gath_a)
        pltpu.sync_copy(gath_a, part_out.at[c, pl.ds(r, ECH)])
        return 0
    lax.fori_loop(0, ROWS_PER_TILE // ECH, dump, 0)


_spmm = pl.kernel(
    _spmm_body,
    out_type=jax.ShapeDtypeStruct((NC, ACC_N, D), jnp.float32),
    mesh=_MESH,
    scratch_types=[
        pltpu.VMEM_SHARED((ACC_N, D), jnp.float32),
        pltpu.VMEM((GE, 2, ECH), jnp.int32),
        pltpu.VMEM((GE, 2, ECH), jnp.int32),
        pltpu.VMEM((GE, 1, ECH), jnp.float32),
        pltpu.VMEM((GE, 1, ECH), jnp.float32),
        pltpu.VMEM((ECH, D), jnp.float32),
        pltpu.VMEM((ECH, D), jnp.float32),
        pltpu.VMEM((ECH, D), jnp.float32),
        pltpu.VMEM((ECH, D), jnp.float32),
        pltpu.SemaphoreType.DMA,
        pltpu.SemaphoreType.DMA,
        pltpu.SemaphoreType.DMA,
        pltpu.SemaphoreType.DMA,
        pltpu.SemaphoreType.DMA,
        pltpu.SemaphoreType.DMA,
        pltpu.SemaphoreType.DMA,
        pltpu.SemaphoreType.DMA,
        pltpu.SemaphoreType.DMA,
    ],
)


def _bpr_body(xf_hbm, emb_hbm, users_hbm, pos_hbm, neg_hbm,
              dif_out, reg_out,
              uidx_v, pidx_v, nidx_v,
              ubuf, pbuf, nbuf, eubuf, epbuf, enbuf,
              dif_v, reg_v, sem):
    c = lax.axis_index("c")
    s = lax.axis_index("s")
    wid = c * NS + s

    pltpu.sync_copy(users_hbm.at[wid, 0], uidx_v)
    pltpu.sync_copy(pos_hbm.at[wid, 0], pidx_v)
    pltpu.sync_copy(neg_hbm.at[wid, 0], nidx_v)

    def offs(i, _):
        sl = pl.ds(i * LANES, LANES)
        pidx_v[sl] = pidx_v[sl] + N_USERS
        nidx_v[sl] = nidx_v[sl] + N_USERS
        return 0
    lax.fori_loop(0, CH // LANES, offs, 0)

    for k in range(CH // SUB):
        ssl = pl.ds(k * SUB, SUB)
        pltpu.async_copy(xf_hbm.at[uidx_v.at[ssl]], ubuf, sem).wait()
        pltpu.async_copy(xf_hbm.at[pidx_v.at[ssl]], pbuf, sem).wait()
        pltpu.async_copy(xf_hbm.at[nidx_v.at[ssl]], nbuf, sem).wait()
        pltpu.async_copy(emb_hbm.at[uidx_v.at[ssl]], eubuf, sem).wait()
        pltpu.async_copy(emb_hbm.at[pidx_v.at[ssl]], epbuf, sem).wait()
        pltpu.async_copy(emb_hbm.at[nidx_v.at[ssl]], enbuf, sem).wait()

        def sample(e, _):
            acc_d = jnp.zeros((LANES,), jnp.float32)
            acc_r = jnp.zeros((LANES,), jnp.float32)
            for d in range(D // LANES):
                sl = pl.ds(d * LANES, LANES)
                u = ubuf[e, sl]
                acc_d = acc_d + u * (pbuf[e, sl] - nbuf[e, sl])
                eu = eubuf[e, sl]
                ep = epbuf[e, sl]
                en = enbuf[e, sl]
                acc_r = acc_r + eu * eu + ep * ep + en * en
            eg = k * SUB + e
            r = eg // 8
            col = (eg % 8) * LANES
            dif_v[r, pl.ds(col, LANES)] = acc_d
            reg_v[r, pl.ds(col, LANES)] = acc_r
            return 0
        lax.fori_loop(0, SUB, sample, 0)

    pltpu.sync_copy(dif_v, dif_out.at[wid])
    pltpu.sync_copy(reg_v, reg_out.at[wid])


_bpr = pl.kernel(
    _bpr_body,
    out_type=(jax.ShapeDtypeStruct((NW, CH * LANES // D, D), jnp.float32),
              jax.ShapeDtypeStruct((NW, CH * LANES // D, D), jnp.float32)),
    mesh=_MESH,
    scratch_types=[
        pltpu.VMEM((CH,), jnp.int32),
        pltpu.VMEM((CH,), jnp.int32),
        pltpu.VMEM((CH,), jnp.int32),
        pltpu.VMEM((SUB, D), jnp.float32),
        pltpu.VMEM((SUB, D), jnp.float32),
        pltpu.VMEM((SUB, D), jnp.float32),
        pltpu.VMEM((SUB, D), jnp.float32),
        pltpu.VMEM((SUB, D), jnp.float32),
        pltpu.VMEM((SUB, D), jnp.float32),
        pltpu.VMEM((CH * LANES // D, D), jnp.float32),
        pltpu.VMEM((CH * LANES // D, D), jnp.float32),
        pltpu.SemaphoreType.DMA,
    ],
)

_ROWS_BLK = 1000


def _combine_body(part_ref, xsum_ref, xnext_ref, xsumo_ref):
    p = part_ref[0] + part_ref[1]
    xnext_ref[...] = p
    xsumo_ref[...] = xsum_ref[...] + p


_combine = pl.pallas_call(
    _combine_body,
    grid=(N_NODES // _ROWS_BLK,),
    in_specs=[
        pl.BlockSpec((NC, _ROWS_BLK, D), lambda i: (0, i, 0)),
        pl.BlockSpec((_ROWS_BLK, D), lambda i: (i, 0)),
    ],
    out_specs=[
        pl.BlockSpec((_ROWS_BLK, D), lambda i: (i, 0)),
        pl.BlockSpec((_ROWS_BLK, D), lambda i: (i, 0)),
    ],
    out_shape=[
        jax.ShapeDtypeStruct((N_NODES, D), jnp.float32),
        jax.ShapeDtypeStruct((N_NODES, D), jnp.float32),
    ],
)


def _combine_final_body(part_ref, xsum_ref, xfin_ref):
    xfin_ref[...] = (xsum_ref[...] + part_ref[0] + part_ref[1]) * (
        1.0 / (NUM_LAYERS + 1))


_combine_final = pl.pallas_call(
    _combine_final_body,
    grid=(N_NODES // _ROWS_BLK,),
    in_specs=[
        pl.BlockSpec((NC, _ROWS_BLK, D), lambda i: (0, i, 0)),
        pl.BlockSpec((_ROWS_BLK, D), lambda i: (i, 0)),
    ],
    out_specs=pl.BlockSpec((_ROWS_BLK, D), lambda i: (i, 0)),
    out_shape=jax.ShapeDtypeStruct((N_NODES, D), jnp.float32),
)


def _loss_body(dif_ref, reg_ref, out_ref):
    sc = jnp.sum(dif_ref[...], axis=1, keepdims=True)
    loss = -jnp.log(jax.nn.sigmoid(sc) + 1e-12)
    reg_mean = jnp.sum(reg_ref[...]) / B
    out_ref[0, 0] = jnp.mean(loss) + REG_W * reg_mean


_loss = pl.pallas_call(
    _loss_body,
    out_specs=pl.BlockSpec(memory_space=pltpu.SMEM),
    out_shape=jax.ShapeDtypeStruct((1, 1), jnp.float32),
)


def kernel(emb_weight, adj_values, adj_indices, users, pos_items, neg_items):
    rows = adj_indices[0]
    cols = adj_indices[1]
    pad = NEDGE_PAD - NE
    zi = jnp.zeros((pad,), jnp.int32)

    def _split(flat):
        n0 = 16 * NCH0 * ECH
        p0 = flat[:n0].reshape(16, NCH0, ECH)
        p1 = flat[n0:].reshape(16, NCH1, ECH)
        arr = jnp.zeros((NW, NCMAX, ECH), flat.dtype)
        return arr.at[:16, :NCH0].set(p0).at[16:, :NCH1].set(p1)

    rows_p = _split(jnp.concatenate([rows, zi]))
    cols_p = _split(jnp.concatenate([cols, zi]))
    vals_p = _split(jnp.concatenate(
        [adj_values, jnp.zeros((pad,), jnp.float32)])).reshape(
            NW, NCMAX, 1, ECH)
    edat = jnp.stack([rows_p, cols_p], axis=2)
    users_r = users.reshape(NW, 1, CH)
    pos_r = pos_items.reshape(NW, 1, CH)
    neg_r = neg_items.reshape(NW, 1, CH)

    part = _spmm(emb_weight, edat, vals_p)
    x1, xsum = _combine(part, emb_weight)
    part = _spmm(x1, edat, vals_p)
    x2, xsum = _combine(part, xsum)
    part = _spmm(x2, edat, vals_p)
    xfinal = _combine_final(part, xsum)

    dif, reg = _bpr(xfinal, emb_weight, users_r, pos_r, neg_r)
    dif = dif.reshape(B, LANES)
    reg = reg.reshape(B, LANES)
    return _loss(dif, reg)[0, 0]


# speedup vs baseline: 1.2233x; 1.2233x over previous
import functools

import jax
import jax.numpy as jnp
from jax import lax
from jax.experimental import pallas as pl
from jax.experimental.pallas import tpu as pltpu
from jax.experimental.pallas import tpu_sc as plsc

N_USERS = 5000
N_NODES = 10000
D = 128
NUM_LAYERS = 3
NE = 320000
B = 4096
REG_W = 1e-4

NC = 2
NS = 16
LANES = 16
NW = NC * NS

ECH = 128
GE = 8
NCH0 = 80
NCH1 = 80
NGRP0 = NCH0 // GE
NGRP1 = NCH1 // GE
NCMAX = max(NCH0, NCH1)
NEDGE_PAD = 16 * (NCH0 + NCH1) * ECH
CH = 128
SUB = 64
ACC_N = 10240
ROWS_PER_TILE = ACC_N // NS

_MESH = plsc.VectorSubcoreMesh(
    core_axis_name="c", subcore_axis_name="s", num_cores=NC, num_subcores=NS)


def _spmm_body(x_hbm, edat_hbm, vals_hbm, part_out,
               acc_sh, edat_a, edat_b, vch_a, vch_b, gath_a, gath_b,
               gsem_a, gsem_b, ssem_a, ssem_b, esem):
    c = lax.axis_index("c")
    s = lax.axis_index("s")
    wid = c * NS + s

    def zfill(i, _):
        r = i // (D // LANES)
        col = (i % (D // LANES)) * LANES
        gath_a[r, pl.ds(col, LANES)] = jnp.zeros((LANES,), jnp.float32)
        return 0
    lax.fori_loop(0, ECH * (D // LANES), zfill, 0)
    row0 = s * ROWS_PER_TILE
    for k in range(ROWS_PER_TILE // ECH):
        pltpu.sync_copy(gath_a, acc_sh.at[pl.ds(row0 + k * ECH, ECH)])
    plsc.subcore_barrier()

    pltpu.sync_copy(edat_hbm.at[wid, pl.ds(0, GE)], edat_a)
    pltpu.sync_copy(vals_hbm.at[wid, pl.ds(0, GE)], vch_a)
    pltpu.make_async_copy(x_hbm.at[edat_a.at[0, 1]], gath_a, gsem_a).start()

    ebufs = ((edat_a, vch_a), (edat_b, vch_b))
    gbufs = ((gath_a, gsem_a, ssem_a), (gath_b, gsem_b, ssem_b))
    ngrp = jnp.where(c == 0, NGRP0, NGRP1)

    def gpair_body(g2, _):
        for gp in range(2):
            g = g2 * 2 + gp
            eg, vg = ebufs[gp]
            eg_n, vg_n = ebufs[1 - gp]
            have_next_group = g + 1 < ngrp
            for j in range(GE):
                i = g * GE + j
                gath, gsem, ssem = gbufs[j % 2]
                gath_o, gsem_o, ssem_o = gbufs[1 - (j % 2)]

                if j == 0:
                    @pl.when(g >= 1)
                    def _():
                        pltpu.make_async_copy(
                            gath_o, acc_sh.at[eg_n.at[GE - 1, 0]],
                            ssem_o).wait()
                else:
                    pltpu.make_async_copy(
                        gath_o, acc_sh.at[eg.at[j - 1, 0]], ssem_o).wait()

                if j == 0:
                    @pl.when(have_next_group)
                    def _():
                        gsl = pl.ds((g + 1) * GE, GE)
                        pltpu.make_async_copy(
                            edat_hbm.at[wid, gsl], eg_n, esem).start()
                        pltpu.make_async_copy(
                            vals_hbm.at[wid, gsl], vg_n, esem).start()

                if j < GE - 1:
                    pltpu.make_async_copy(
                        x_hbm.at[eg.at[j + 1, 1]], gath_o, gsem_o).start()
                else:
                    @pl.when(have_next_group)
                    def _():
                        gsl = pl.ds((g + 1) * GE, GE)
                        pltpu.make_async_copy(
                            edat_hbm.at[wid, gsl], eg_n, esem).wait()
                        pltpu.make_async_copy(
                            vals_hbm.at[wid, gsl], vg_n, esem).wait()
                        pltpu.make_async_copy(
                            x_hbm.at[eg_n.at[0, 1]], gath_o, gsem_o).start()

                pltpu.make_async_copy(
                    x_hbm.at[eg.at[j, 1]], gath, gsem).wait()

                def scale(eb, _):
                    val16 = vg[j, 0, pl.ds(eb * LANES, LANES)]
                    for jj in range(LANES):
                        v = val16[jj]
                        e = eb * LANES + jj
                        for d in range(D // LANES):
                            sl = pl.ds(d * LANES, LANES)
                            gath[e, sl] = gath[e, sl] * v
                    return 0
                lax.fori_loop(0, ECH // LANES, scale, 0)

                pltpu.make_async_copy(
                    gath, acc_sh.at[eg.at[j, 0]], ssem).start(add=True)
        return 0
    lax.fori_loop(0, ngrp // 2, gpair_body, 0)

    pltpu.make_async_copy(
        gath_b, acc_sh.at[edat_b.at[GE - 1, 0]], ssem_b).wait()

    plsc.subcore_barrier()
    def dump(k, _):
        r = row0 + k * ECH
        pltpu.sync_copy(acc_sh.at[pl.ds(r, ECH)], gath_a)
        pltpu.sync_copy(gath_a, part_out.at[c, pl.ds(r, ECH)])
        return 0
    lax.fori_loop(0, ROWS_PER_TILE // ECH, dump, 0)


_spmm = pl.kernel(
    _spmm_body,
    out_type=jax.ShapeDtypeStruct((NC, ACC_N, D), jnp.float32),
    mesh=_MESH,
    scratch_types=[
        pltpu.VMEM_SHARED((ACC_N, D), jnp.float32),
        pltpu.VMEM((GE, 2, ECH), jnp.int32),
        pltpu.VMEM((GE, 2, ECH), jnp.int32),
        pltpu.VMEM((GE, 1, ECH), jnp.float32),
        pltpu.VMEM((GE, 1, ECH), jnp.float32),
        pltpu.VMEM((ECH, D), jnp.float32),
        pltpu.VMEM((ECH, D), jnp.float32),
        pltpu.SemaphoreType.DMA,
        pltpu.SemaphoreType.DMA,
        pltpu.SemaphoreType.DMA,
        pltpu.SemaphoreType.DMA,
        pltpu.SemaphoreType.DMA,
    ],
)


def _bpr_body(xf_hbm, emb_hbm, users_hbm, pos_hbm, neg_hbm,
              dif_out, reg_out,
              uidx_v, pidx_v, nidx_v,
              ubuf, pbuf, nbuf, eubuf, epbuf, enbuf,
              dif_v, reg_v, sem):
    c = lax.axis_index("c")
    s = lax.axis_index("s")
    wid = c * NS + s

    pltpu.sync_copy(users_hbm.at[wid, 0], uidx_v)
    pltpu.sync_copy(pos_hbm.at[wid, 0], pidx_v)
    pltpu.sync_copy(neg_hbm.at[wid, 0], nidx_v)

    def offs(i, _):
        sl = pl.ds(i * LANES, LANES)
        pidx_v[sl] = pidx_v[sl] + N_USERS
        nidx_v[sl] = nidx_v[sl] + N_USERS
        return 0
    lax.fori_loop(0, CH // LANES, offs, 0)

    for k in range(CH // SUB):
        ssl = pl.ds(k * SUB, SUB)
        pltpu.async_copy(xf_hbm.at[uidx_v.at[ssl]], ubuf, sem).wait()
        pltpu.async_copy(xf_hbm.at[pidx_v.at[ssl]], pbuf, sem).wait()
        pltpu.async_copy(xf_hbm.at[nidx_v.at[ssl]], nbuf, sem).wait()
        pltpu.async_copy(emb_hbm.at[uidx_v.at[ssl]], eubuf, sem).wait()
        pltpu.async_copy(emb_hbm.at[pidx_v.at[ssl]], epbuf, sem).wait()
        pltpu.async_copy(emb_hbm.at[nidx_v.at[ssl]], enbuf, sem).wait()

        def sample(e, _):
            acc_d = jnp.zeros((LANES,), jnp.float32)
            acc_r = jnp.zeros((LANES,), jnp.float32)
            for d in range(D // LANES):
                sl = pl.ds(d * LANES, LANES)
                u = ubuf[e, sl]
                acc_d = acc_d + u * (pbuf[e, sl] - nbuf[e, sl])
                eu = eubuf[e, sl]
                ep = epbuf[e, sl]
                en = enbuf[e, sl]
                acc_r = acc_r + eu * eu + ep * ep + en * en
            eg = k * SUB + e
            r = eg // 8
            col = (eg % 8) * LANES
            dif_v[r, pl.ds(col, LANES)] = acc_d
            reg_v[r, pl.ds(col, LANES)] = acc_r
            return 0
        lax.fori_loop(0, SUB, sample, 0)

    pltpu.sync_copy(dif_v, dif_out.at[wid])
    pltpu.sync_copy(reg_v, reg_out.at[wid])


_bpr = pl.kernel(
    _bpr_body,
    out_type=(jax.ShapeDtypeStruct((NW, CH * LANES // D, D), jnp.float32),
              jax.ShapeDtypeStruct((NW, CH * LANES // D, D), jnp.float32)),
    mesh=_MESH,
    scratch_types=[
        pltpu.VMEM((CH,), jnp.int32),
        pltpu.VMEM((CH,), jnp.int32),
        pltpu.VMEM((CH,), jnp.int32),
        pltpu.VMEM((SUB, D), jnp.float32),
        pltpu.VMEM((SUB, D), jnp.float32),
        pltpu.VMEM((SUB, D), jnp.float32),
        pltpu.VMEM((SUB, D), jnp.float32),
        pltpu.VMEM((SUB, D), jnp.float32),
        pltpu.VMEM((SUB, D), jnp.float32),
        pltpu.VMEM((CH * LANES // D, D), jnp.float32),
        pltpu.VMEM((CH * LANES // D, D), jnp.float32),
        pltpu.SemaphoreType.DMA,
    ],
)

_ROWS_BLK = 1000


def _combine_body(part_ref, xsum_ref, xnext_ref, xsumo_ref):
    p = part_ref[0] + part_ref[1]
    xnext_ref[...] = p
    xsumo_ref[...] = xsum_ref[...] + p


_combine = pl.pallas_call(
    _combine_body,
    grid=(N_NODES // _ROWS_BLK,),
    in_specs=[
        pl.BlockSpec((NC, _ROWS_BLK, D), lambda i: (0, i, 0)),
        pl.BlockSpec((_ROWS_BLK, D), lambda i: (i, 0)),
    ],
    out_specs=[
        pl.BlockSpec((_ROWS_BLK, D), lambda i: (i, 0)),
        pl.BlockSpec((_ROWS_BLK, D), lambda i: (i, 0)),
    ],
    out_shape=[
        jax.ShapeDtypeStruct((N_NODES, D), jnp.float32),
        jax.ShapeDtypeStruct((N_NODES, D), jnp.float32),
    ],
)


def _combine_final_body(part_ref, xsum_ref, xfin_ref):
    xfin_ref[...] = (xsum_ref[...] + part_ref[0] + part_ref[1]) * (
        1.0 / (NUM_LAYERS + 1))


_combine_final = pl.pallas_call(
    _combine_final_body,
    grid=(N_NODES // _ROWS_BLK,),
    in_specs=[
        pl.BlockSpec((NC, _ROWS_BLK, D), lambda i: (0, i, 0)),
        pl.BlockSpec((_ROWS_BLK, D), lambda i: (i, 0)),
    ],
    out_specs=pl.BlockSpec((_ROWS_BLK, D), lambda i: (i, 0)),
    out_shape=jax.ShapeDtypeStruct((N_NODES, D), jnp.float32),
)


def _loss_body(dif_ref, reg_ref, out_ref):
    sc = jnp.sum(dif_ref[...], axis=1, keepdims=True)
    loss = -jnp.log(jax.nn.sigmoid(sc) + 1e-12)
    reg_mean = jnp.sum(reg_ref[...]) / B
    out_ref[0, 0] = jnp.mean(loss) + REG_W * reg_mean


_loss = pl.pallas_call(
    _loss_body,
    out_specs=pl.BlockSpec(memory_space=pltpu.SMEM),
    out_shape=jax.ShapeDtypeStruct((1, 1), jnp.float32),
)


def kernel(emb_weight, adj_values, adj_indices, users, pos_items, neg_items):
    rows = adj_indices[0]
    cols = adj_indices[1]
    pad = NEDGE_PAD - NE
    zi = jnp.zeros((pad,), jnp.int32)

    def _split(flat):
        n0 = 16 * NCH0 * ECH
        p0 = flat[:n0].reshape(16, NCH0, ECH)
        p1 = flat[n0:].reshape(16, NCH1, ECH)
        arr = jnp.zeros((NW, NCMAX, ECH), flat.dtype)
        return arr.at[:16, :NCH0].set(p0).at[16:, :NCH1].set(p1)

    rows_p = _split(jnp.concatenate([rows, zi]))
    cols_p = _split(jnp.concatenate([cols, zi]))
    vals_p = _split(jnp.concatenate(
        [adj_values, jnp.zeros((pad,), jnp.float32)])).reshape(
            NW, NCMAX, 1, ECH)
    edat = jnp.stack([rows_p, cols_p], axis=2)
    users_r = users.reshape(NW, 1, CH)
    pos_r = pos_items.reshape(NW, 1, CH)
    neg_r = neg_items.reshape(NW, 1, CH)

    part = _spmm(emb_weight, edat, vals_p)
    x1, xsum = _combine(part, emb_weight)
    part = _spmm(x1, edat, vals_p)
    x2, xsum = _combine(part, xsum)
    part = _spmm(x2, edat, vals_p)
    xfinal = _combine_final(part, xsum)

    dif, reg = _bpr(xfinal, emb_weight, users_r, pos_r, neg_r)
    dif = dif.reshape(B, LANES)
    reg = reg.reshape(B, LANES)
    return _loss(dif, reg)[0, 0]
